# Initial kernel scaffold; baseline (speedup 1.0000x reference)
import jax
import jax.numpy as jnp
from jax.experimental import pallas as pl


def kernel(x, edge_index, head, tail, input, W_gnn, W_self, b_gnn, W1, b1, W2, b2, W3, b3):
    raise NotImplementedError("write your pallas kernel here")


# speedup vs baseline: 7.2496x; 7.2496x over previous
import functools

import jax
import jax.numpy as jnp
from jax import lax
from jax.experimental import pallas as pl
from jax.experimental.pallas import tpu as pltpu
from jax.experimental.pallas import tpu_sc as plsc

NC = 2
NS = 16
NW = NC * NS

_f32 = jnp.float32


def _make_agg_kernel(N, D, E):
    EPW = E // NW
    C = 80
    NCH = EPW // C
    ZR = (N // NS) // 8 * 8
    TAIL = N - NS * ZR
    mesh = plsc.VectorSubcoreMesh(core_axis_name="c", subcore_axis_name="s",
                                  num_cores=NC, num_subcores=NS)

    @functools.partial(
        pl.kernel, mesh=mesh,
        out_type=[jax.ShapeDtypeStruct((N, D), _f32),
                  jax.ShapeDtypeStruct((N, D), _f32),
                  jax.ShapeDtypeStruct((N,), _f32),
                  jax.ShapeDtypeStruct((N,), _f32)],
        scratch_types=[pltpu.VMEM((NCH, C), jnp.int32),
                       pltpu.VMEM((NCH, C), jnp.int32),
                       pltpu.VMEM((C,), _f32),
                       pltpu.VMEM((C, D), _f32),
                       pltpu.VMEM_SHARED((N, D), _f32),
                       pltpu.VMEM_SHARED((N,), _f32),
                       pltpu.SemaphoreType.DMA],
    )
    def agg_kernel(x_hbm, src_hbm, dst_hbm, za_hbm, zd_hbm,
                   a0, a1, d0, d1,
                   src_v, dst_v, ones_v, rows_v, agg_sh, deg_sh, gsem):
        cid = lax.axis_index("c")
        sid = lax.axis_index("s")
        wid = sid * NC + cid
        for k in range(C // 16):
            ones_v[pl.ds(k * 16, 16)] = jnp.ones((16,), _f32)
        pltpu.sync_copy(src_hbm.at[wid], src_v)
        pltpu.sync_copy(dst_hbm.at[wid], dst_v)
        pltpu.sync_copy(za_hbm.at[pl.ds(sid * ZR, ZR), :],
                        agg_sh.at[pl.ds(sid * ZR, ZR), :])

        @pl.when(sid == 0)
        def _():
            pltpu.sync_copy(za_hbm.at[pl.ds(NS * ZR, TAIL), :],
                            agg_sh.at[pl.ds(NS * ZR, TAIL), :])
            pltpu.sync_copy(zd_hbm, deg_sh)

        plsc.subcore_barrier()

        def chunk(j, carry):
            pltpu.async_copy(x_hbm.at[src_v.at[j]], rows_v, gsem).wait()
            pltpu.sync_copy(rows_v, agg_sh.at[dst_v.at[j]], add=True)
            pltpu.sync_copy(ones_v, deg_sh.at[dst_v.at[j]], add=True)
            return carry

        lax.fori_loop(0, NCH, chunk, 0)
        plsc.subcore_barrier()

        @pl.when(cid == 0)
        def _():
            pltpu.sync_copy(agg_sh.at[pl.ds(sid * ZR, ZR), :],
                            a0.at[pl.ds(sid * ZR, ZR), :])

        @pl.when(cid == 1)
        def _():
            pltpu.sync_copy(agg_sh.at[pl.ds(sid * ZR, ZR), :],
                            a1.at[pl.ds(sid * ZR, ZR), :])

        @pl.when(jnp.logical_and(cid == 0, sid == 0))
        def _():
            pltpu.sync_copy(agg_sh.at[pl.ds(NS * ZR, TAIL), :],
                            a0.at[pl.ds(NS * ZR, TAIL), :])
            pltpu.sync_copy(deg_sh, d0)

        @pl.when(jnp.logical_and(cid == 1, sid == 0))
        def _():
            pltpu.sync_copy(agg_sh.at[pl.ds(NS * ZR, TAIL), :],
                            a1.at[pl.ds(NS * ZR, TAIL), :])
            pltpu.sync_copy(deg_sh, d1)

    return agg_kernel


def _make_pair_gather_kernel(N, D, B2):
    PPT = B2 // NW
    GC = 128
    NJ = PPT // GC
    mesh = plsc.VectorSubcoreMesh(core_axis_name="c", subcore_axis_name="s",
                                  num_cores=NC, num_subcores=NS)

    @functools.partial(
        pl.kernel, mesh=mesh,
        out_type=[jax.ShapeDtypeStruct((B2, D), _f32),
                  jax.ShapeDtypeStruct((B2, D), _f32),
                  jax.ShapeDtypeStruct((B2, D), _f32),
                  jax.ShapeDtypeStruct((B2,), _f32),
                  jax.ShapeDtypeStruct((B2,), _f32)],
        scratch_types=[pltpu.VMEM((NJ, GC), jnp.int32),
                       pltpu.VMEM((GC, D), _f32),
                       pltpu.VMEM((GC, D), _f32),
                       pltpu.VMEM((GC, D), _f32),
                       pltpu.VMEM((GC,), _f32),
                       pltpu.VMEM((GC,), _f32),
                       pltpu.SemaphoreType.DMA],
    )
    def pair_kernel(a0_hbm, a1_hbm, x_hbm, d0_hbm, d1_hbm, hp_hbm,
                    ga0, ga1, gx, gd0, gd1,
                    idx_v, r0, r1, rx, s0, s1, gsem):
        cid = lax.axis_index("c")
        sid = lax.axis_index("s")
        wid = sid * NC + cid
        pltpu.sync_copy(hp_hbm.at[wid], idx_v)
        for j in range(NJ):
            base = wid * PPT + j * GC
            pltpu.async_copy(a0_hbm.at[idx_v.at[j]], r0, gsem).wait()
            pltpu.sync_copy(r0, ga0.at[pl.ds(base, GC), :])
            pltpu.async_copy(a1_hbm.at[idx_v.at[j]], r1, gsem).wait()
            pltpu.sync_copy(r1, ga1.at[pl.ds(base, GC), :])
            pltpu.async_copy(x_hbm.at[idx_v.at[j]], rx, gsem).wait()
            pltpu.sync_copy(rx, gx.at[pl.ds(base, GC), :])
            pltpu.async_copy(d0_hbm.at[idx_v.at[j]], s0, gsem).wait()
            pltpu.sync_copy(s0, gd0.at[pl.ds(base, GC)])
            pltpu.async_copy(d1_hbm.at[idx_v.at[j]], s1, gsem).wait()
            pltpu.sync_copy(s1, gd1.at[pl.ds(base, GC)])

    return pair_kernel


def _mlp_body(a0h, a0t, a1h, a1t, xh, xt, dh0, dh1, dt0, dt1,
              wg, ws, bg, w1h, w1t, b1r, w2, b2r, w3r, b3r, out_ref):
    def node_repr(a0, a1, xg, da, db):
        agg = a0[...] + a1[...]
        deg = da[...] + db[...]
        s = agg / jnp.maximum(deg, 1.0)
        z = (jnp.dot(s, wg[...], preferred_element_type=_f32)
             + jnp.dot(xg[...], ws[...], preferred_element_type=_f32)
             + bg[...])
        return jnp.maximum(z, 0.0)

    rh = node_repr(a0h, a1h, xh, dh0, dh1)
    rt = node_repr(a0t, a1t, xt, dt0, dt1)
    h = jnp.maximum(jnp.dot(rh, w1h[...], preferred_element_type=_f32)
                    + jnp.dot(rt, w1t[...], preferred_element_type=_f32)
                    + b1r[...], 0.0)
    h = jnp.maximum(jnp.dot(h, w2[...], preferred_element_type=_f32)
                    + b2r[...], 0.0)
    out_ref[...] = jnp.sum(h * w3r[...], axis=1, keepdims=True) + b3r[...]


def _mlp_call(B, D, ga0, ga1, gx, gd0, gd1,
              W_gnn, W_self, bg, W1h, W1t, b1r, W2, b2r, W3r, b3r):
    BLK = 1024
    G = B // BLK
    row_h = pl.BlockSpec((BLK, D), lambda i: (i, 0))
    row_t = pl.BlockSpec((BLK, D), lambda i: (i + G, 0))
    deg_h = pl.BlockSpec((BLK, 1), lambda i: (i, 0))
    deg_t = pl.BlockSpec((BLK, 1), lambda i: (i + G, 0))

    def full(a):
        return pl.BlockSpec(a.shape, lambda i: tuple(0 for _ in a.shape))

    gd0c = gd0.reshape(2 * B, 1)
    gd1c = gd1.reshape(2 * B, 1)
    return pl.pallas_call(
        _mlp_body,
        grid=(G,),
        in_specs=[row_h, row_t, row_h, row_t, row_h, row_t,
                  deg_h, deg_h, deg_t, deg_t,
                  full(W_gnn), full(W_self), full(bg),
                  full(W1h), full(W1t), full(b1r),
                  full(W2), full(b2r), full(W3r), full(b3r)],
        out_specs=pl.BlockSpec((BLK, 1), lambda i: (i, 0)),
        out_shape=jax.ShapeDtypeStruct((B, 1), _f32),
    )(ga0, ga0, ga1, ga1, gx, gx, gd0c, gd1c, gd0c, gd1c,
      W_gnn, W_self, bg, W1h, W1t, b1r, W2, b2r, W3r, b3r)


def kernel(x, edge_index, head, tail, input, W_gnn, W_self, b_gnn,
           W1, b1, W2, b2, W3, b3):
    N, D = x.shape
    E = edge_index.shape[1]
    B = head.shape[0]
    C = 80
    assert E % (NW * C) == 0 and N % NS == 0 and (2 * B) % (NW * 128) == 0

    src2 = edge_index[0].reshape(NW, E // (NW * C), C)
    dst2 = edge_index[1].reshape(NW, E // (NW * C), C)
    za = jnp.zeros((N, D), _f32)
    zd = jnp.zeros((N,), _f32)
    a0, a1, d0, d1 = _make_agg_kernel(N, D, E)(x, src2, dst2, za, zd)

    hp = jnp.concatenate([head, tail]).reshape(NW, (2 * B) // (NW * 128), 128)
    ga0, ga1, gx, gd0, gd1 = _make_pair_gather_kernel(N, D, 2 * B)(
        a0, a1, x, d0, d1, hp)

    bg = b_gnn.reshape(1, D)
    b1r = b1.reshape(1, D)
    b2r = b2.reshape(1, D)
    W3r = W3.reshape(1, D)
    b3r = b3.reshape(1, 1)
    return _mlp_call(B, D, ga0, ga1, gx, gd0, gd1,
                     W_gnn, W_self, bg, W1[:D], W1[D:], b1r, W2, b2r, W3r, b3r)


# speedup vs baseline: 10.8763x; 1.5003x over previous
import functools

import jax
import jax.numpy as jnp
from jax import lax
from jax.experimental import pallas as pl
from jax.experimental.pallas import tpu as pltpu
from jax.experimental.pallas import tpu_sc as plsc

NC = 2
NS = 16
NW = NC * NS

_f32 = jnp.float32


def _make_agg_kernel(N, D, E):
    EPW = E // NW
    C = 80
    NCH = EPW // C
    ZR = (N // NS) // 8 * 8
    TAIL = N - NS * ZR
    mesh = plsc.VectorSubcoreMesh(core_axis_name="c", subcore_axis_name="s",
                                  num_cores=NC, num_subcores=NS)

    @functools.partial(
        pl.kernel, mesh=mesh,
        out_type=[jax.ShapeDtypeStruct((N, D), _f32),
                  jax.ShapeDtypeStruct((N, D), _f32),
                  jax.ShapeDtypeStruct((N,), _f32),
                  jax.ShapeDtypeStruct((N,), _f32)],
        scratch_types=[pltpu.VMEM((EPW,), jnp.int32),
                       pltpu.VMEM((NCH, C), jnp.int32),
                       pltpu.VMEM((C,), _f32),
                       pltpu.VMEM((2, C, D), _f32),
                       pltpu.VMEM_SHARED((N, D), _f32),
                       pltpu.VMEM_SHARED((N,), _f32),
                       pltpu.SemaphoreType.DMA],
    )
    def agg_kernel(x_hbm, src_hbm, dst_hbm, za_hbm, zd_hbm,
                   a0, a1, d0, d1,
                   src_v, dst_v, ones_v, rows_v, agg_sh, deg_sh, gsem):
        cid = lax.axis_index("c")
        sid = lax.axis_index("s")
        wid = sid * NC + cid
        for k in range(C // 16):
            ones_v[pl.ds(k * 16, 16)] = jnp.ones((16,), _f32)
        pltpu.sync_copy(src_hbm.at[pl.ds(wid * EPW, EPW)], src_v)
        pltpu.sync_copy(dst_hbm.at[wid], dst_v)
        pltpu.sync_copy(za_hbm.at[pl.ds(sid * ZR, ZR), :],
                        agg_sh.at[pl.ds(sid * ZR, ZR), :])

        @pl.when(sid == 0)
        def _():
            pltpu.sync_copy(za_hbm.at[pl.ds(NS * ZR, TAIL), :],
                            agg_sh.at[pl.ds(NS * ZR, TAIL), :])
            pltpu.sync_copy(zd_hbm, deg_sh)

        plsc.subcore_barrier()

        def src_idx(j):
            return src_v.at[pl.ds(pl.multiple_of(j * C, C), C)]

        pltpu.async_copy(x_hbm.at[src_idx(0)], rows_v.at[0], gsem)

        def chunk(j, carry):
            @pl.when(j + 1 < NCH)
            def _():
                pltpu.async_copy(x_hbm.at[src_idx(j + 1)],
                                 rows_v.at[(j + 1) % 2], gsem)

            pltpu.make_async_copy(x_hbm.at[src_idx(j)],
                                  rows_v.at[j % 2], gsem).wait()
            pltpu.sync_copy(rows_v.at[j % 2], agg_sh.at[dst_v.at[j]], add=True)
            pltpu.sync_copy(ones_v, deg_sh.at[dst_v.at[j]], add=True)
            return carry

        lax.fori_loop(0, NCH, chunk, 0)
        plsc.subcore_barrier()

        @pl.when(cid == 0)
        def _():
            pltpu.sync_copy(agg_sh.at[pl.ds(sid * ZR, ZR), :],
                            a0.at[pl.ds(sid * ZR, ZR), :])

        @pl.when(cid == 1)
        def _():
            pltpu.sync_copy(agg_sh.at[pl.ds(sid * ZR, ZR), :],
                            a1.at[pl.ds(sid * ZR, ZR), :])

        @pl.when(jnp.logical_and(cid == 0, sid == 0))
        def _():
            pltpu.sync_copy(agg_sh.at[pl.ds(NS * ZR, TAIL), :],
                            a0.at[pl.ds(NS * ZR, TAIL), :])
            pltpu.sync_copy(deg_sh, d0)

        @pl.when(jnp.logical_and(cid == 1, sid == 0))
        def _():
            pltpu.sync_copy(agg_sh.at[pl.ds(NS * ZR, TAIL), :],
                            a1.at[pl.ds(NS * ZR, TAIL), :])
            pltpu.sync_copy(deg_sh, d1)

    return agg_kernel


def _make_pair_gather_kernel(N, D, B2):
    PPT = B2 // NW
    GC = 128
    NJ = PPT // GC
    mesh = plsc.VectorSubcoreMesh(core_axis_name="c", subcore_axis_name="s",
                                  num_cores=NC, num_subcores=NS)

    @functools.partial(
        pl.kernel, mesh=mesh,
        out_type=[jax.ShapeDtypeStruct((B2, D), _f32),
                  jax.ShapeDtypeStruct((B2, D), _f32),
                  jax.ShapeDtypeStruct((B2, D), _f32),
                  jax.ShapeDtypeStruct((B2,), _f32),
                  jax.ShapeDtypeStruct((B2,), _f32)],
        scratch_types=[pltpu.VMEM((NJ, GC), jnp.int32),
                       pltpu.VMEM((2, GC, D), _f32),
                       pltpu.VMEM((2, GC, D), _f32),
                       pltpu.VMEM((2, GC, D), _f32),
                       pltpu.VMEM((2, GC), _f32),
                       pltpu.VMEM((2, GC), _f32),
                       pltpu.SemaphoreType.DMA],
    )
    def pair_kernel(a0_hbm, a1_hbm, x_hbm, d0_hbm, d1_hbm, hp_hbm,
                    ga0, ga1, gx, gd0, gd1,
                    idx_v, r0, r1, rx, s0, s1, gsem):
        cid = lax.axis_index("c")
        sid = lax.axis_index("s")
        wid = sid * NC + cid
        pltpu.sync_copy(hp_hbm.at[wid], idx_v)

        def fire(j, b):
            idx = idx_v.at[j]
            pltpu.async_copy(a0_hbm.at[idx], r0.at[b], gsem)
            pltpu.async_copy(a1_hbm.at[idx], r1.at[b], gsem)
            pltpu.async_copy(x_hbm.at[idx], rx.at[b], gsem)
            pltpu.async_copy(d0_hbm.at[idx], s0.at[b], gsem)
            pltpu.async_copy(d1_hbm.at[idx], s1.at[b], gsem)

        def drain(j, b):
            idx = idx_v.at[j]
            pltpu.make_async_copy(a0_hbm.at[idx], r0.at[b], gsem).wait()
            pltpu.make_async_copy(a1_hbm.at[idx], r1.at[b], gsem).wait()
            pltpu.make_async_copy(x_hbm.at[idx], rx.at[b], gsem).wait()
            pltpu.make_async_copy(d0_hbm.at[idx], s0.at[b], gsem).wait()
            pltpu.make_async_copy(d1_hbm.at[idx], s1.at[b], gsem).wait()

        fire(0, 0)
        for j in range(NJ):
            if j + 1 < NJ:
                fire(j + 1, (j + 1) % 2)
            drain(j, j % 2)
            base = wid * PPT + j * GC
            pltpu.sync_copy(r0.at[j % 2], ga0.at[pl.ds(base, GC), :])
            pltpu.sync_copy(r1.at[j % 2], ga1.at[pl.ds(base, GC), :])
            pltpu.sync_copy(rx.at[j % 2], gx.at[pl.ds(base, GC), :])
            pltpu.sync_copy(s0.at[j % 2], gd0.at[pl.ds(base, GC)])
            pltpu.sync_copy(s1.at[j % 2], gd1.at[pl.ds(base, GC)])

    return pair_kernel


def _mlp_body(a0h, a0t, a1h, a1t, xh, xt, dh0, dh1, dt0, dt1,
              wg, ws, bg, w1h, w1t, b1r, w2, b2r, w3r, b3r, out_ref):
    def node_repr(a0, a1, xg, da, db):
        agg = a0[...] + a1[...]
        deg = da[...] + db[...]
        s = agg / jnp.maximum(deg, 1.0)
        z = (jnp.dot(s, wg[...], preferred_element_type=_f32)
             + jnp.dot(xg[...], ws[...], preferred_element_type=_f32)
             + bg[...])
        return jnp.maximum(z, 0.0)

    rh = node_repr(a0h, a1h, xh, dh0, dh1)
    rt = node_repr(a0t, a1t, xt, dt0, dt1)
    h = jnp.maximum(jnp.dot(rh, w1h[...], preferred_element_type=_f32)
                    + jnp.dot(rt, w1t[...], preferred_element_type=_f32)
                    + b1r[...], 0.0)
    h = jnp.maximum(jnp.dot(h, w2[...], preferred_element_type=_f32)
                    + b2r[...], 0.0)
    out_ref[...] = jnp.sum(h * w3r[...], axis=1, keepdims=True) + b3r[...]


def _mlp_call(B, D, ga0, ga1, gx, gd0, gd1,
              W_gnn, W_self, bg, W1h, W1t, b1r, W2, b2r, W3r, b3r):
    BLK = 1024
    G = B // BLK
    row_h = pl.BlockSpec((BLK, D), lambda i: (i, 0))
    row_t = pl.BlockSpec((BLK, D), lambda i: (i + G, 0))
    deg_h = pl.BlockSpec((BLK, 1), lambda i: (i, 0))
    deg_t = pl.BlockSpec((BLK, 1), lambda i: (i + G, 0))

    def full(a):
        return pl.BlockSpec(a.shape, lambda i: tuple(0 for _ in a.shape))

    gd0c = gd0.reshape(2 * B, 1)
    gd1c = gd1.reshape(2 * B, 1)
    return pl.pallas_call(
        _mlp_body,
        grid=(G,),
        in_specs=[row_h, row_t, row_h, row_t, row_h, row_t,
                  deg_h, deg_h, deg_t, deg_t,
                  full(W_gnn), full(W_self), full(bg),
                  full(W1h), full(W1t), full(b1r),
                  full(W2), full(b2r), full(W3r), full(b3r)],
        out_specs=pl.BlockSpec((BLK, 1), lambda i: (i, 0)),
        out_shape=jax.ShapeDtypeStruct((B, 1), _f32),
    )(ga0, ga0, ga1, ga1, gx, gx, gd0c, gd1c, gd0c, gd1c,
      W_gnn, W_self, bg, W1h, W1t, b1r, W2, b2r, W3r, b3r)


def kernel(x, edge_index, head, tail, input, W_gnn, W_self, b_gnn,
           W1, b1, W2, b2, W3, b3):
    N, D = x.shape
    E = edge_index.shape[1]
    B = head.shape[0]
    C = 80
    assert E % (NW * C) == 0 and N % NS == 0 and (2 * B) % (NW * 128) == 0

    src2 = edge_index[0]
    dst2 = edge_index[1].reshape(NW, E // (NW * C), C)
    za = jnp.zeros((N, D), _f32)
    zd = jnp.zeros((N,), _f32)
    a0, a1, d0, d1 = _make_agg_kernel(N, D, E)(x, src2, dst2, za, zd)

    hp = jnp.concatenate([head, tail]).reshape(NW, (2 * B) // (NW * 128), 128)
    ga0, ga1, gx, gd0, gd1 = _make_pair_gather_kernel(N, D, 2 * B)(
        a0, a1, x, d0, d1, hp)

    bg = b_gnn.reshape(1, D)
    b1r = b1.reshape(1, D)
    b2r = b2.reshape(1, D)
    W3r = W3.reshape(1, D)
    b3r = b3.reshape(1, 1)
    return _mlp_call(B, D, ga0, ga1, gx, gd0, gd1,
                     W_gnn, W_self, bg, W1[:D], W1[D:], b1r, W2, b2r, W3r, b3r)


# speedup vs baseline: 10.9965x; 1.0110x over previous
import functools

import jax
import jax.numpy as jnp
from jax import lax
from jax.experimental import pallas as pl
from jax.experimental.pallas import tpu as pltpu
from jax.experimental.pallas import tpu_sc as plsc

NC = 2
NS = 16
NW = NC * NS

_f32 = jnp.float32


def _make_agg_kernel(N, D, E):
    EPW = E // NW
    C = 80
    NCH = EPW // C
    ZR = (N // NS) // 8 * 8
    TAIL = N - NS * ZR
    mesh = plsc.VectorSubcoreMesh(core_axis_name="c", subcore_axis_name="s",
                                  num_cores=NC, num_subcores=NS)

    @functools.partial(
        pl.kernel, mesh=mesh,
        out_type=[jax.ShapeDtypeStruct((N, D), _f32),
                  jax.ShapeDtypeStruct((N, D), _f32),
                  jax.ShapeDtypeStruct((N,), _f32),
                  jax.ShapeDtypeStruct((N,), _f32)],
        scratch_types=[pltpu.VMEM((EPW,), jnp.int32),
                       pltpu.VMEM((NCH, C), jnp.int32),
                       pltpu.VMEM((C,), _f32),
                       pltpu.VMEM((2, C, D), _f32),
                       pltpu.VMEM_SHARED((N, D), _f32),
                       pltpu.VMEM_SHARED((N,), _f32),
                       pltpu.SemaphoreType.DMA,
                       pltpu.SemaphoreType.DMA,
                       pltpu.SemaphoreType.DMA],
    )
    def agg_kernel(x_hbm, src_hbm, dst_hbm, za_hbm, zd_hbm,
                   a0, a1, d0, d1,
                   src_v, dst_v, ones_v, rows_v, agg_sh, deg_sh,
                   gsem, ssem, dsem):
        cid = lax.axis_index("c")
        sid = lax.axis_index("s")
        wid = sid * NC + cid
        for k in range(C // 16):
            ones_v[pl.ds(k * 16, 16)] = jnp.ones((16,), _f32)
        pltpu.sync_copy(src_hbm.at[pl.ds(wid * EPW, EPW)], src_v)
        pltpu.sync_copy(dst_hbm.at[wid], dst_v)
        pltpu.sync_copy(za_hbm.at[pl.ds(sid * ZR, ZR), :],
                        agg_sh.at[pl.ds(sid * ZR, ZR), :])

        @pl.when(sid == 0)
        def _():
            pltpu.sync_copy(za_hbm.at[pl.ds(NS * ZR, TAIL), :],
                            agg_sh.at[pl.ds(NS * ZR, TAIL), :])
            pltpu.sync_copy(zd_hbm, deg_sh)

        plsc.subcore_barrier()

        def src_idx(j):
            return src_v.at[pl.ds(pl.multiple_of(j * C, C), C)]

        pltpu.async_copy(x_hbm.at[src_idx(0)], rows_v.at[0], gsem)

        def chunk(j, carry):
            @pl.when(j >= 1)
            def _():
                pltpu.make_async_copy(rows_v.at[(j - 1) % 2],
                                      agg_sh.at[dst_v.at[j - 1]], ssem).wait()
                pltpu.make_async_copy(ones_v,
                                      deg_sh.at[dst_v.at[j - 1]], dsem).wait()

            @pl.when(j + 1 < NCH)
            def _():
                pltpu.async_copy(x_hbm.at[src_idx(j + 1)],
                                 rows_v.at[(j + 1) % 2], gsem)

            pltpu.make_async_copy(x_hbm.at[src_idx(j)],
                                  rows_v.at[j % 2], gsem).wait()
            pltpu.async_copy(rows_v.at[j % 2], agg_sh.at[dst_v.at[j]],
                             ssem, add=True)
            pltpu.async_copy(ones_v, deg_sh.at[dst_v.at[j]], dsem, add=True)
            return carry

        lax.fori_loop(0, NCH, chunk, 0)
        pltpu.make_async_copy(rows_v.at[(NCH - 1) % 2],
                              agg_sh.at[dst_v.at[NCH - 1]], ssem).wait()
        pltpu.make_async_copy(ones_v, deg_sh.at[dst_v.at[NCH - 1]],
                              dsem).wait()
        plsc.subcore_barrier()

        @pl.when(cid == 0)
        def _():
            pltpu.sync_copy(agg_sh.at[pl.ds(sid * ZR, ZR), :],
                            a0.at[pl.ds(sid * ZR, ZR), :])

        @pl.when(cid == 1)
        def _():
            pltpu.sync_copy(agg_sh.at[pl.ds(sid * ZR, ZR), :],
                            a1.at[pl.ds(sid * ZR, ZR), :])

        @pl.when(jnp.logical_and(cid == 0, sid == 0))
        def _():
            pltpu.sync_copy(agg_sh.at[pl.ds(NS * ZR, TAIL), :],
                            a0.at[pl.ds(NS * ZR, TAIL), :])
            pltpu.sync_copy(deg_sh, d0)

        @pl.when(jnp.logical_and(cid == 1, sid == 0))
        def _():
            pltpu.sync_copy(agg_sh.at[pl.ds(NS * ZR, TAIL), :],
                            a1.at[pl.ds(NS * ZR, TAIL), :])
            pltpu.sync_copy(deg_sh, d1)

    return agg_kernel


def _make_pair_gather_kernel(N, D, B2):
    PPT = B2 // NW
    GC = 128
    NJ = PPT // GC
    mesh = plsc.VectorSubcoreMesh(core_axis_name="c", subcore_axis_name="s",
                                  num_cores=NC, num_subcores=NS)

    @functools.partial(
        pl.kernel, mesh=mesh,
        out_type=[jax.ShapeDtypeStruct((B2, D), _f32),
                  jax.ShapeDtypeStruct((B2, D), _f32),
                  jax.ShapeDtypeStruct((B2, D), _f32),
                  jax.ShapeDtypeStruct((B2,), _f32),
                  jax.ShapeDtypeStruct((B2,), _f32)],
        scratch_types=[pltpu.VMEM((NJ, GC), jnp.int32),
                       pltpu.VMEM((2, GC, D), _f32),
                       pltpu.VMEM((2, GC, D), _f32),
                       pltpu.VMEM((2, GC, D), _f32),
                       pltpu.VMEM((2, GC), _f32),
                       pltpu.VMEM((2, GC), _f32),
                       pltpu.SemaphoreType.DMA],
    )
    def pair_kernel(a0_hbm, a1_hbm, x_hbm, d0_hbm, d1_hbm, hp_hbm,
                    ga0, ga1, gx, gd0, gd1,
                    idx_v, r0, r1, rx, s0, s1, gsem):
        cid = lax.axis_index("c")
        sid = lax.axis_index("s")
        wid = sid * NC + cid
        pltpu.sync_copy(hp_hbm.at[wid], idx_v)

        def fire(j, b):
            idx = idx_v.at[j]
            pltpu.async_copy(a0_hbm.at[idx], r0.at[b], gsem)
            pltpu.async_copy(a1_hbm.at[idx], r1.at[b], gsem)
            pltpu.async_copy(x_hbm.at[idx], rx.at[b], gsem)
            pltpu.async_copy(d0_hbm.at[idx], s0.at[b], gsem)
            pltpu.async_copy(d1_hbm.at[idx], s1.at[b], gsem)

        def drain(j, b):
            idx = idx_v.at[j]
            pltpu.make_async_copy(a0_hbm.at[idx], r0.at[b], gsem).wait()
            pltpu.make_async_copy(a1_hbm.at[idx], r1.at[b], gsem).wait()
            pltpu.make_async_copy(x_hbm.at[idx], rx.at[b], gsem).wait()
            pltpu.make_async_copy(d0_hbm.at[idx], s0.at[b], gsem).wait()
            pltpu.make_async_copy(d1_hbm.at[idx], s1.at[b], gsem).wait()

        fire(0, 0)
        for j in range(NJ):
            if j + 1 < NJ:
                fire(j + 1, (j + 1) % 2)
            drain(j, j % 2)
            base = wid * PPT + j * GC
            pltpu.sync_copy(r0.at[j % 2], ga0.at[pl.ds(base, GC), :])
            pltpu.sync_copy(r1.at[j % 2], ga1.at[pl.ds(base, GC), :])
            pltpu.sync_copy(rx.at[j % 2], gx.at[pl.ds(base, GC), :])
            pltpu.sync_copy(s0.at[j % 2], gd0.at[pl.ds(base, GC)])
            pltpu.sync_copy(s1.at[j % 2], gd1.at[pl.ds(base, GC)])

    return pair_kernel


def _mlp_body(a0h, a0t, a1h, a1t, xh, xt, dh0, dh1, dt0, dt1,
              wg, ws, bg, w1h, w1t, b1r, w2, b2r, w3r, b3r, out_ref):
    def node_repr(a0, a1, xg, da, db):
        agg = a0[...] + a1[...]
        deg = da[...] + db[...]
        s = agg / jnp.maximum(deg, 1.0)
        z = (jnp.dot(s, wg[...], preferred_element_type=_f32)
             + jnp.dot(xg[...], ws[...], preferred_element_type=_f32)
             + bg[...])
        return jnp.maximum(z, 0.0)

    rh = node_repr(a0h, a1h, xh, dh0, dh1)
    rt = node_repr(a0t, a1t, xt, dt0, dt1)
    h = jnp.maximum(jnp.dot(rh, w1h[...], preferred_element_type=_f32)
                    + jnp.dot(rt, w1t[...], preferred_element_type=_f32)
                    + b1r[...], 0.0)
    h = jnp.maximum(jnp.dot(h, w2[...], preferred_element_type=_f32)
                    + b2r[...], 0.0)
    out_ref[...] = jnp.sum(h * w3r[...], axis=1, keepdims=True) + b3r[...]


def _mlp_call(B, D, ga0, ga1, gx, gd0, gd1,
              W_gnn, W_self, bg, W1h, W1t, b1r, W2, b2r, W3r, b3r):
    BLK = 1024
    G = B // BLK
    row_h = pl.BlockSpec((BLK, D), lambda i: (i, 0))
    row_t = pl.BlockSpec((BLK, D), lambda i: (i + G, 0))
    deg_h = pl.BlockSpec((BLK, 1), lambda i: (i, 0))
    deg_t = pl.BlockSpec((BLK, 1), lambda i: (i + G, 0))

    def full(a):
        return pl.BlockSpec(a.shape, lambda i: tuple(0 for _ in a.shape))

    gd0c = gd0.reshape(2 * B, 1)
    gd1c = gd1.reshape(2 * B, 1)
    return pl.pallas_call(
        _mlp_body,
        grid=(G,),
        in_specs=[row_h, row_t, row_h, row_t, row_h, row_t,
                  deg_h, deg_h, deg_t, deg_t,
                  full(W_gnn), full(W_self), full(bg),
                  full(W1h), full(W1t), full(b1r),
                  full(W2), full(b2r), full(W3r), full(b3r)],
        out_specs=pl.BlockSpec((BLK, 1), lambda i: (i, 0)),
        out_shape=jax.ShapeDtypeStruct((B, 1), _f32),
    )(ga0, ga0, ga1, ga1, gx, gx, gd0c, gd1c, gd0c, gd1c,
      W_gnn, W_self, bg, W1h, W1t, b1r, W2, b2r, W3r, b3r)


def kernel(x, edge_index, head, tail, input, W_gnn, W_self, b_gnn,
           W1, b1, W2, b2, W3, b3):
    N, D = x.shape
    E = edge_index.shape[1]
    B = head.shape[0]
    C = 80
    assert E % (NW * C) == 0 and N % NS == 0 and (2 * B) % (NW * 128) == 0

    src2 = edge_index[0]
    dst2 = edge_index[1].reshape(NW, E // (NW * C), C)
    za = jnp.zeros((N, D), _f32)
    zd = jnp.zeros((N,), _f32)
    a0, a1, d0, d1 = _make_agg_kernel(N, D, E)(x, src2, dst2, za, zd)

    hp = jnp.concatenate([head, tail]).reshape(NW, (2 * B) // (NW * 128), 128)
    ga0, ga1, gx, gd0, gd1 = _make_pair_gather_kernel(N, D, 2 * B)(
        a0, a1, x, d0, d1, hp)

    bg = b_gnn.reshape(1, D)
    b1r = b1.reshape(1, D)
    b2r = b2.reshape(1, D)
    W3r = W3.reshape(1, D)
    b3r = b3.reshape(1, 1)
    return _mlp_call(B, D, ga0, ga1, gx, gd0, gd1,
                     W_gnn, W_self, bg, W1[:D], W1[D:], b1r, W2, b2r, W3r, b3r)
